# Initial kernel scaffold; baseline (speedup 1.0000x reference)
import jax
import jax.numpy as jnp
from jax.experimental import pallas as pl


def kernel(x, emb_weight):
    raise NotImplementedError("write your pallas kernel here")


# speedup vs baseline: 3.0264x; 3.0264x over previous
import jax
import jax.numpy as jnp
from jax.experimental import pallas as pl


def _copy_block(in_ref, out_ref):
    out_ref[...] = in_ref[...]


def kernel(x, emb_weight):
    sl = x.shape[1]
    model_dim = emb_weight.shape[1]
    block_rows = 1024
    num_blocks = sl // block_rows
    return pl.pallas_call(
        _copy_block,
        grid=(num_blocks,),
        in_specs=[pl.BlockSpec((block_rows, model_dim), lambda i: (i, 0))],
        out_specs=pl.BlockSpec((block_rows, model_dim), lambda i: (i, 0)),
        out_shape=jax.ShapeDtypeStruct((sl, model_dim), emb_weight.dtype),
    )(emb_weight)
